# baseline (speedup 1.0000x reference)
import functools

import jax
import jax.numpy as jnp
from jax import lax
from jax.experimental import pallas as pl
from jax.experimental.pallas import tpu as pltpu
from jax.experimental.pallas import tpu_sc as plsc

HX = 2.0 / 2047.0
HY = 1.0 / 1023.0
H = W = 512
NPIX = H * W
NI = 7
NB = 4
B = 2048
NPLANE = 2 * NI
TBL = NPLANE * B

_INTERPRET = False
_DIAG = 0


def _pass1_body(pred_ref, inst_ref, lab_ref, out_ref):
    r = pl.program_id(1)
    sigma = pred_ref[0, 0]
    seed = jax.nn.sigmoid(pred_ref[0, 1])
    inst = inst_ref[0]
    lab = lab_ref[0]
    rows = sigma.shape[0]
    row0 = (r * rows).astype(jnp.float32)
    xm = lax.broadcasted_iota(jnp.int32, sigma.shape, 1).astype(jnp.float32) * HX
    ym = (lax.broadcasted_iota(jnp.int32, sigma.shape, 0).astype(jnp.float32) + row0) * HY

    io = lax.broadcasted_iota(jnp.int32, (1, 128), 1)
    bg = jnp.sum(jnp.where(lab == 0, seed * seed, 0.0))
    zero = jnp.zeros((1, 128), jnp.float32)
    cntv, sxv, syv, ssv, ss2v = zero, zero, zero, zero, zero
    bgv = jnp.where(io == 0, bg, 0.0)
    for i in range(NI):
        mf = (inst == (i + 1)).astype(jnp.float32)
        sel = (io == i)
        cntv = cntv + jnp.where(sel, jnp.sum(mf), 0.0)
        sxv = sxv + jnp.where(sel, jnp.sum(mf * xm), 0.0)
        syv = syv + jnp.where(sel, jnp.sum(mf * ym), 0.0)
        ssv = ssv + jnp.where(sel, jnp.sum(mf * sigma), 0.0)
        ss2v = ss2v + jnp.where(sel, jnp.sum(mf * sigma * sigma), 0.0)
    acc = jnp.concatenate([cntv, sxv, syv, ssv, ss2v, bgv], axis=0)

    @pl.when(r == 0)
    def _():
        out_ref[0] = acc

    @pl.when(r != 0)
    def _():
        out_ref[0] = out_ref[0] + acc


def _pass1(prediction, instances, labels):
    rows = 256
    nr = H // rows
    return pl.pallas_call(
        _pass1_body,
        grid=(NB, nr),
        in_specs=[
            pl.BlockSpec((1, 2, rows, W), lambda b, r: (b, 1, r, 0)),
            pl.BlockSpec((1, rows, W), lambda b, r: (b, r, 0)),
            pl.BlockSpec((1, rows, W), lambda b, r: (b, r, 0)),
        ],
        out_specs=pl.BlockSpec((1, 6, 128), lambda b, r: (b, 0, 0)),
        out_shape=jax.ShapeDtypeStruct((NB, 6, 128), jnp.float32),
        interpret=_INTERPRET,
    )(prediction, instances, labels)


def _pass2_body(bo, scal_ref, pred_ref, inst_ref, idx_ref, sfg_ref):
    b = pl.program_id(0) + bo
    k = pl.program_id(1)
    p = pred_ref[0]
    rows = p.shape[1]
    row0 = (k * rows).astype(jnp.float32)
    xm = lax.broadcasted_iota(jnp.int32, (rows, W), 1).astype(jnp.float32) * HX
    ym = (lax.broadcasted_iota(jnp.int32, (rows, W), 0).astype(jnp.float32) + row0) * HY
    ex = jnp.tanh(p[0]) + xm
    ey = jnp.tanh(p[1]) + ym
    sig = p[2]
    seed = jax.nn.sigmoid(p[3])
    inst = inst_ref[0]

    io = lax.broadcasted_iota(jnp.int32, (1, 128), 1)
    sacc = jnp.zeros((1, 128), jnp.float32)
    bf = jnp.float32(B)
    for i in range(NI):
        safe = jnp.maximum(scal_ref[b, 0, i], 1.0)
        cx = scal_ref[b, 1, i] / safe
        cy = scal_ref[b, 2, i] / safe
        s = jnp.exp(10.0 * scal_ref[b, 3, i] / safe)
        dx = ex - cx
        dy = ey - cy
        d = jnp.exp(-(dx * dx + dy * dy) * s)
        own = inst == (i + 1)
        jp = jnp.clip((bf * (1.0 - d)).astype(jnp.int32), 0, B - 1)
        jn = jnp.clip((bf * d).astype(jnp.int32), 0, B - 1)
        idx_ref[0, i] = jnp.where(own, i * 2 * B + jp, (i * 2 + 1) * B + jn)
        sfg = jnp.sum(jnp.where(own, (seed - d) ** 2, 0.0))
        sacc = sacc + jnp.where(io == i, sfg, 0.0)

    @pl.when(k == 0)
    def _():
        sfg_ref[0] = sacc

    @pl.when(k != 0)
    def _():
        sfg_ref[0] = sfg_ref[0] + sacc


def _pass2(scal, prediction, instances, bo, nb):
    rows = 64
    nk = H // rows
    return pl.pallas_call(
        functools.partial(_pass2_body, bo),
        grid=(nb, nk),
        in_specs=[
            pl.BlockSpec(memory_space=pltpu.SMEM),
            pl.BlockSpec((1, 4, rows, W), lambda b, k: (b + bo, 0, k, 0)),
            pl.BlockSpec((1, rows, W), lambda b, k: (b + bo, k, 0)),
        ],
        out_specs=[
            pl.BlockSpec((1, NI, rows, W), lambda b, k: (b, 0, k, 0)),
            pl.BlockSpec((1, 1, 128), lambda b, k: (b, 0, 0)),
        ],
        out_shape=[
            jax.ShapeDtypeStruct((nb, NI, H, W), jnp.int32),
            jax.ShapeDtypeStruct((nb, 1, 128), jnp.float32),
        ],
        interpret=_INTERPRET,
    )(scal, prediction, instances)


def _pass3_body(*refs):
    nsplit = (len(refs) - 2) // 2
    parts_refs = refs[:nsplit]
    sums_ref = refs[nsplit]
    sfg_refs = refs[nsplit + 1:2 * nsplit + 1]
    out_ref = refs[2 * nsplit + 1]
    per = NB // nsplit
    iar = lax.broadcasted_iota(jnp.int32, (B, B), 0)
    iac = lax.broadcasted_iota(jnp.int32, (B, B), 1)
    M = (iar >= iac).astype(jnp.float32)
    total = jnp.float32(0.0)
    for b in range(NB):
        parts_ref = parts_refs[b // per]
        sfg_ref = sfg_refs[b // per]
        bl = b % per
        tb = jnp.sum(parts_ref[bl], axis=0)
        suf = jnp.dot(tb, M, preferred_element_type=jnp.float32)
        inst_loss = jnp.float32(0.0)
        var_loss = jnp.float32(0.0)
        obj = jnp.float32(0.0)
        seed_fg = jnp.float32(0.0)
        for i in range(NI):
            G = sums_ref[b, 0, i]
            pres = (G > 0.0).astype(jnp.float32)
            Gs = jnp.maximum(G, 1.0)
            C = suf[2 * i:2 * i + 1]
            Nn = suf[2 * i + 1:2 * i + 2]
            Nt = C + Nn
            J = 1.0 - (G - C) / jnp.maximum(G + Nt - C, 1.0)
            lov = (2.0 / B) * (jnp.sum(J) - 0.5 * J[0, 0])
            inst_loss = inst_loss + pres * lov
            ss = sums_ref[b, 3, i]
            ss2 = sums_ref[b, 4, i]
            mu = ss / Gs
            var_loss = var_loss + pres * (ss2 / Gs - mu * mu)
            seed_fg = seed_fg + pres * sfg_ref[bl, i]
            obj = obj + pres
        denom = jnp.maximum(obj, 1.0)
        bg = sums_ref[b, 5, 0]
        seed_loss = (bg + seed_fg) / jnp.float32(NPIX)
        total = total + inst_loss / denom + 10.0 * var_loss / denom + seed_loss
    out_ref[0, 0] = total / NB


def _pass3(parts_list, sums, sfg_list):
    n = len(parts_list)
    return pl.pallas_call(
        _pass3_body,
        in_specs=(
            [pl.BlockSpec(memory_space=pltpu.VMEM)] * n
            + [pl.BlockSpec(memory_space=pltpu.SMEM)] * (n + 1)
        ),
        out_specs=pl.BlockSpec(memory_space=pltpu.SMEM),
        out_shape=jax.ShapeDtypeStruct((1, 1), jnp.float32),
        interpret=_INTERPRET,
    )(*parts_list, sums, *sfg_list)


NW = 32
CHROWS = 8
CHUNK = CHROWS * W


def _sc_hist_body(nbatch, idx_hbm, out_hbm, buf0, buf1, table, sem0, sem1):
    slots = NW // nbatch
    rows_per_slot = H // slots
    nch = NI * rows_per_slot // CHROWS
    cid = lax.axis_index("c")
    sid = lax.axis_index("s")
    wid = sid * 2 + cid
    batch = wid // slots
    slot = wid - batch * slots
    base_row = slot * rows_per_slot

    zeros = jnp.zeros((16,), jnp.float32)
    ones = jnp.ones((16,), jnp.float32)

    @plsc.parallel_loop(0, TBL // 16, unroll=8)
    def _(j):
        table[pl.ds(j * 16, 16)] = zeros

    def start(c, buf, sem):
        i = c // (rows_per_slot // CHROWS)
        rb = c - i * (rows_per_slot // CHROWS)
        pltpu.async_copy(
            idx_hbm.at[batch, i, pl.ds(base_row + rb * CHROWS, CHROWS)],
            buf, sem)

    def wait(buf, sem):
        pltpu.make_async_copy(
            idx_hbm.at[batch, 0, pl.ds(0, CHROWS)], buf, sem).wait()

    def process(buf):
        @plsc.parallel_loop(0, CHUNK // 16, unroll=8)
        def _(j):
            v = buf[j >> 5, pl.ds((j & 31) * 16, 16)]
            plsc.addupdate_scatter(table, [v], ones)

    start(0, buf0, sem0)

    def pair_body(p, carry):
        c0 = p * 2
        start(c0 + 1, buf1, sem1)
        wait(buf0, sem0)
        process(buf0)

        @pl.when(c0 + 2 < nch)
        def _():
            start(c0 + 2, buf0, sem0)
        wait(buf1, sem1)
        process(buf1)
        return carry
    lax.fori_loop(0, nch // 2, pair_body, 0)
    pltpu.sync_copy(table, out_hbm.at[wid])


def _sc_hist(idx_half, nbatch):
    mesh = plsc.VectorSubcoreMesh(core_axis_name="c", subcore_axis_name="s")
    f = functools.partial(
        pl.kernel,
        mesh=mesh,
        compiler_params=pltpu.CompilerParams(needs_layout_passes=False),
        out_type=jax.ShapeDtypeStruct((NW, TBL), jnp.float32),
        scratch_types=[
            pltpu.VMEM((CHROWS, W), jnp.int32),
            pltpu.VMEM((CHROWS, W), jnp.int32),
            pltpu.VMEM((TBL,), jnp.float32),
            pltpu.SemaphoreType.DMA,
            pltpu.SemaphoreType.DMA,
        ],
    )(functools.partial(_sc_hist_body, nbatch))
    return f(idx_half)


def kernel(prediction, instances, labels):
    sums = _pass1(prediction, instances, labels)
    nsplit = 4
    per = NB // nsplit
    parts_list, sfg_list = [], []
    for g in range(nsplit):
        idxg, sfgg = _pass2(sums, prediction, instances, g * per, per)
        parts_list.append(
            _sc_hist(idxg, per).reshape(per, NW // per, NPLANE, B))
        sfg_list.append(sfgg[:, 0, :NI])

    out = _pass3(parts_list, sums, sfg_list)
    return out.reshape(())


# speedup vs baseline: 1.0623x; 1.0623x over previous
import functools

import jax
import jax.numpy as jnp
from jax import lax
from jax.experimental import pallas as pl
from jax.experimental.pallas import tpu as pltpu
from jax.experimental.pallas import tpu_sc as plsc

HX = 2.0 / 2047.0
HY = 1.0 / 1023.0
H = W = 512
NPIX = H * W
NI = 7
NB = 4
B = 2048
NPLANE = 2 * NI
TBL = NPLANE * B

_INTERPRET = False
_DIAG = 0


def _pass1_body(pred_ref, inst_ref, lab_ref, out_ref):
    r = pl.program_id(1)
    sigma = pred_ref[0, 0]
    seed = jax.nn.sigmoid(pred_ref[0, 1])
    inst = inst_ref[0]
    lab = lab_ref[0]
    rows = sigma.shape[0]
    row0 = (r * rows).astype(jnp.float32)
    xm = lax.broadcasted_iota(jnp.int32, sigma.shape, 1).astype(jnp.float32) * HX
    ym = (lax.broadcasted_iota(jnp.int32, sigma.shape, 0).astype(jnp.float32) + row0) * HY

    io = lax.broadcasted_iota(jnp.int32, (1, 128), 1)
    bg = jnp.sum(jnp.where(lab == 0, seed * seed, 0.0))
    zero = jnp.zeros((1, 128), jnp.float32)
    cntv, sxv, syv, ssv, ss2v = zero, zero, zero, zero, zero
    bgv = jnp.where(io == 0, bg, 0.0)
    for i in range(NI):
        mf = (inst == (i + 1)).astype(jnp.float32)
        sel = (io == i)
        cntv = cntv + jnp.where(sel, jnp.sum(mf), 0.0)
        sxv = sxv + jnp.where(sel, jnp.sum(mf * xm), 0.0)
        syv = syv + jnp.where(sel, jnp.sum(mf * ym), 0.0)
        ssv = ssv + jnp.where(sel, jnp.sum(mf * sigma), 0.0)
        ss2v = ss2v + jnp.where(sel, jnp.sum(mf * sigma * sigma), 0.0)
    acc = jnp.concatenate([cntv, sxv, syv, ssv, ss2v, bgv], axis=0)

    @pl.when(r == 0)
    def _():
        out_ref[0] = acc

    @pl.when(r != 0)
    def _():
        out_ref[0] = out_ref[0] + acc


def _pass1(prediction, instances, labels):
    rows = 256
    nr = H // rows
    return pl.pallas_call(
        _pass1_body,
        grid=(NB, nr),
        in_specs=[
            pl.BlockSpec((1, 2, rows, W), lambda b, r: (b, 1, r, 0)),
            pl.BlockSpec((1, rows, W), lambda b, r: (b, r, 0)),
            pl.BlockSpec((1, rows, W), lambda b, r: (b, r, 0)),
        ],
        out_specs=pl.BlockSpec((1, 6, 128), lambda b, r: (b, 0, 0)),
        out_shape=jax.ShapeDtypeStruct((NB, 6, 128), jnp.float32),
        interpret=_INTERPRET,
    )(prediction, instances, labels)


def _pass2_body(bo, scal_ref, pred_ref, inst_ref, idx_ref, sfg_ref):
    b = pl.program_id(0) + bo
    k = pl.program_id(1)
    p = pred_ref[0]
    rows = p.shape[1]
    row0 = (k * rows).astype(jnp.float32)
    xm = lax.broadcasted_iota(jnp.int32, (rows, W), 1).astype(jnp.float32) * HX
    ym = (lax.broadcasted_iota(jnp.int32, (rows, W), 0).astype(jnp.float32) + row0) * HY
    ex = jnp.tanh(p[0]) + xm
    ey = jnp.tanh(p[1]) + ym
    sig = p[2]
    seed = jax.nn.sigmoid(p[3])
    inst = inst_ref[0]

    io = lax.broadcasted_iota(jnp.int32, (1, 128), 1)
    sacc = jnp.zeros((1, 128), jnp.float32)
    bf = jnp.float32(B)
    for i in range(NI):
        safe = jnp.maximum(scal_ref[b, 0, i], 1.0)
        cx = scal_ref[b, 1, i] / safe
        cy = scal_ref[b, 2, i] / safe
        s = jnp.exp(10.0 * scal_ref[b, 3, i] / safe)
        dx = ex - cx
        dy = ey - cy
        d = jnp.exp(-(dx * dx + dy * dy) * s)
        own = inst == (i + 1)
        jp = jnp.clip((bf * (1.0 - d)).astype(jnp.int32), 0, B - 1)
        jn = jnp.clip((bf * d).astype(jnp.int32), 0, B - 1)
        idx_ref[0, i] = jnp.where(own, i * 2 * B + jp, (i * 2 + 1) * B + jn)
        sfg = jnp.sum(jnp.where(own, (seed - d) ** 2, 0.0))
        sacc = sacc + jnp.where(io == i, sfg, 0.0)

    @pl.when(k == 0)
    def _():
        sfg_ref[0] = sacc

    @pl.when(k != 0)
    def _():
        sfg_ref[0] = sfg_ref[0] + sacc


def _pass2(scal, prediction, instances, bo, nb):
    rows = 64
    nk = H // rows
    return pl.pallas_call(
        functools.partial(_pass2_body, bo),
        grid=(nb, nk),
        in_specs=[
            pl.BlockSpec(memory_space=pltpu.SMEM),
            pl.BlockSpec((1, 4, rows, W), lambda b, k: (b + bo, 0, k, 0)),
            pl.BlockSpec((1, rows, W), lambda b, k: (b + bo, k, 0)),
        ],
        out_specs=[
            pl.BlockSpec((1, NI, rows, W), lambda b, k: (b, 0, k, 0)),
            pl.BlockSpec((1, 1, 128), lambda b, k: (b, 0, 0)),
        ],
        out_shape=[
            jax.ShapeDtypeStruct((nb, NI, H, W), jnp.int32),
            jax.ShapeDtypeStruct((nb, 1, 128), jnp.float32),
        ],
        interpret=_INTERPRET,
    )(scal, prediction, instances)


def _pass3_body(*refs):
    nsplit = (len(refs) - 2) // 2
    parts_refs = refs[:nsplit]
    sums_ref = refs[nsplit]
    sfg_refs = refs[nsplit + 1:2 * nsplit + 1]
    out_ref = refs[2 * nsplit + 1]
    per = NB // nsplit
    iar = lax.broadcasted_iota(jnp.int32, (B, B), 0)
    iac = lax.broadcasted_iota(jnp.int32, (B, B), 1)
    M = (iar >= iac).astype(jnp.float32)
    total = jnp.float32(0.0)
    for b in range(NB):
        parts_ref = parts_refs[b // per]
        sfg_ref = sfg_refs[b // per]
        bl = b % per
        tb = jnp.sum(parts_ref[bl], axis=0)
        suf = jnp.dot(tb, M, preferred_element_type=jnp.float32)
        inst_loss = jnp.float32(0.0)
        var_loss = jnp.float32(0.0)
        obj = jnp.float32(0.0)
        seed_fg = jnp.float32(0.0)
        for i in range(NI):
            G = sums_ref[b, 0, i]
            pres = (G > 0.0).astype(jnp.float32)
            Gs = jnp.maximum(G, 1.0)
            C = suf[2 * i:2 * i + 1]
            Nn = suf[2 * i + 1:2 * i + 2]
            Nt = C + Nn
            J = 1.0 - (G - C) / jnp.maximum(G + Nt - C, 1.0)
            lov = (2.0 / B) * (jnp.sum(J) - 0.5 * J[0, 0])
            inst_loss = inst_loss + pres * lov
            ss = sums_ref[b, 3, i]
            ss2 = sums_ref[b, 4, i]
            mu = ss / Gs
            var_loss = var_loss + pres * (ss2 / Gs - mu * mu)
            seed_fg = seed_fg + pres * sfg_ref[bl, i]
            obj = obj + pres
        denom = jnp.maximum(obj, 1.0)
        bg = sums_ref[b, 5, 0]
        seed_loss = (bg + seed_fg) / jnp.float32(NPIX)
        total = total + inst_loss / denom + 10.0 * var_loss / denom + seed_loss
    out_ref[0, 0] = total / NB


def _pass3(parts_list, sums, sfg_list):
    n = len(parts_list)
    return pl.pallas_call(
        _pass3_body,
        in_specs=(
            [pl.BlockSpec(memory_space=pltpu.VMEM)] * n
            + [pl.BlockSpec(memory_space=pltpu.SMEM)] * (n + 1)
        ),
        out_specs=pl.BlockSpec(memory_space=pltpu.SMEM),
        out_shape=jax.ShapeDtypeStruct((1, 1), jnp.float32),
        interpret=_INTERPRET,
    )(*parts_list, sums, *sfg_list)


NW = 32
CHROWS = 16
CHUNK = CHROWS * W


def _sc_hist_body(nbatch, idx_hbm, out_hbm, buf0, buf1, table, sem0, sem1):
    slots = NW // nbatch
    rows_per_slot = H // slots
    nch = NI * rows_per_slot // CHROWS
    cid = lax.axis_index("c")
    sid = lax.axis_index("s")
    wid = sid * 2 + cid
    batch = wid // slots
    slot = wid - batch * slots
    base_row = slot * rows_per_slot

    zeros = jnp.zeros((16,), jnp.float32)
    ones = jnp.ones((16,), jnp.float32)

    @plsc.parallel_loop(0, TBL // 16, unroll=8)
    def _(j):
        table[pl.ds(j * 16, 16)] = zeros

    def start(c, buf, sem):
        i = c // (rows_per_slot // CHROWS)
        rb = c - i * (rows_per_slot // CHROWS)
        pltpu.async_copy(
            idx_hbm.at[batch, i, pl.ds(base_row + rb * CHROWS, CHROWS)],
            buf, sem)

    def wait(buf, sem):
        pltpu.make_async_copy(
            idx_hbm.at[batch, 0, pl.ds(0, CHROWS)], buf, sem).wait()

    def process(buf):
        @plsc.parallel_loop(0, CHUNK // 16, unroll=8)
        def _(j):
            v = buf[j >> 5, pl.ds((j & 31) * 16, 16)]
            plsc.addupdate_scatter(table, [v], ones)

    start(0, buf0, sem0)

    def pair_body(p, carry):
        c0 = p * 2
        start(c0 + 1, buf1, sem1)
        wait(buf0, sem0)
        process(buf0)

        @pl.when(c0 + 2 < nch)
        def _():
            start(c0 + 2, buf0, sem0)
        wait(buf1, sem1)
        process(buf1)
        return carry
    lax.fori_loop(0, nch // 2, pair_body, 0)
    pltpu.sync_copy(table, out_hbm.at[wid])


def _sc_hist(idx_half, nbatch):
    mesh = plsc.VectorSubcoreMesh(core_axis_name="c", subcore_axis_name="s")
    f = functools.partial(
        pl.kernel,
        mesh=mesh,
        compiler_params=pltpu.CompilerParams(needs_layout_passes=False),
        out_type=jax.ShapeDtypeStruct((NW, TBL), jnp.float32),
        scratch_types=[
            pltpu.VMEM((CHROWS, W), jnp.int32),
            pltpu.VMEM((CHROWS, W), jnp.int32),
            pltpu.VMEM((TBL,), jnp.float32),
            pltpu.SemaphoreType.DMA,
            pltpu.SemaphoreType.DMA,
        ],
    )(functools.partial(_sc_hist_body, nbatch))
    return f(idx_half)


def kernel(prediction, instances, labels):
    sums = _pass1(prediction, instances, labels)
    nsplit = 2
    per = NB // nsplit
    parts_list, sfg_list = [], []
    for g in range(nsplit):
        idxg, sfgg = _pass2(sums, prediction, instances, g * per, per)
        parts_list.append(
            _sc_hist(idxg, per).reshape(per, NW // per, NPLANE, B))
        sfg_list.append(sfgg[:, 0, :NI])

    out = _pass3(parts_list, sums, sfg_list)
    return out.reshape(())


# speedup vs baseline: 1.1447x; 1.0776x over previous
import functools

import jax
import jax.numpy as jnp
from jax import lax
from jax.experimental import pallas as pl
from jax.experimental.pallas import tpu as pltpu
from jax.experimental.pallas import tpu_sc as plsc

HX = 2.0 / 2047.0
HY = 1.0 / 1023.0
H = W = 512
NPIX = H * W
NI = 7
NB = 4
B = 2048
NPLANE = 2 * NI
TBL = NPLANE * B

_INTERPRET = False
_DIAG = 0


def _pass1_body(pred_ref, inst_ref, lab_ref, out_ref):
    r = pl.program_id(1)
    sigma = pred_ref[0, 0]
    seed = jax.nn.sigmoid(pred_ref[0, 1])
    inst = inst_ref[0]
    lab = lab_ref[0]
    rows = sigma.shape[0]
    row0 = (r * rows).astype(jnp.float32)
    xm = lax.broadcasted_iota(jnp.int32, sigma.shape, 1).astype(jnp.float32) * HX
    ym = (lax.broadcasted_iota(jnp.int32, sigma.shape, 0).astype(jnp.float32) + row0) * HY

    io = lax.broadcasted_iota(jnp.int32, (1, 128), 1)
    bg = jnp.sum(jnp.where(lab == 0, seed * seed, 0.0))
    zero = jnp.zeros((1, 128), jnp.float32)
    cntv, sxv, syv, ssv, ss2v = zero, zero, zero, zero, zero
    bgv = jnp.where(io == 0, bg, 0.0)
    for i in range(NI):
        mf = (inst == (i + 1)).astype(jnp.float32)
        sel = (io == i)
        cntv = cntv + jnp.where(sel, jnp.sum(mf), 0.0)
        sxv = sxv + jnp.where(sel, jnp.sum(mf * xm), 0.0)
        syv = syv + jnp.where(sel, jnp.sum(mf * ym), 0.0)
        ssv = ssv + jnp.where(sel, jnp.sum(mf * sigma), 0.0)
        ss2v = ss2v + jnp.where(sel, jnp.sum(mf * sigma * sigma), 0.0)
    acc = jnp.concatenate([cntv, sxv, syv, ssv, ss2v, bgv], axis=0)

    @pl.when(r == 0)
    def _():
        out_ref[0] = acc

    @pl.when(r != 0)
    def _():
        out_ref[0] = out_ref[0] + acc


def _pass1(prediction, instances, labels):
    rows = 256
    nr = H // rows
    return pl.pallas_call(
        _pass1_body,
        grid=(NB, nr),
        in_specs=[
            pl.BlockSpec((1, 2, rows, W), lambda b, r: (b, 1, r, 0)),
            pl.BlockSpec((1, rows, W), lambda b, r: (b, r, 0)),
            pl.BlockSpec((1, rows, W), lambda b, r: (b, r, 0)),
        ],
        out_specs=pl.BlockSpec((1, 6, 128), lambda b, r: (b, 0, 0)),
        out_shape=jax.ShapeDtypeStruct((NB, 6, 128), jnp.float32),
        interpret=_INTERPRET,
    )(prediction, instances, labels)


def _pass2_body(bo, scal_ref, pred_ref, inst_ref, idx_ref, sfg_ref):
    b = pl.program_id(0) + bo
    k = pl.program_id(1)
    p = pred_ref[0]
    rows = p.shape[1]
    row0 = (k * rows).astype(jnp.float32)
    xm = lax.broadcasted_iota(jnp.int32, (rows, W), 1).astype(jnp.float32) * HX
    ym = (lax.broadcasted_iota(jnp.int32, (rows, W), 0).astype(jnp.float32) + row0) * HY
    ex = jnp.tanh(p[0]) + xm
    ey = jnp.tanh(p[1]) + ym
    sig = p[2]
    seed = jax.nn.sigmoid(p[3])
    inst = inst_ref[0]

    io = lax.broadcasted_iota(jnp.int32, (1, 128), 1)
    bf = jnp.float32(B)
    r2p = ex * ex + ey * ey
    down = jnp.zeros_like(ex)
    for i in range(NI):
        safe = jnp.maximum(scal_ref[b, 0, i], 1.0)
        cx = scal_ref[b, 1, i] / safe
        cy = scal_ref[b, 2, i] / safe
        s = jnp.exp(10.0 * scal_ref[b, 3, i] / safe)
        bx = -2.0 * s * cx
        by = -2.0 * s * cy
        c0 = s * (cx * cx + cy * cy)
        u = s * r2p + bx * ex + by * ey + c0
        d = jnp.exp(-u)
        own = inst == (i + 1)
        jn = jnp.minimum((bf * d).astype(jnp.int32), B - 1)
        idx_ref[0, i] = jnp.where(own, (i * 2 * B + B - 1) - jn,
                                  (i * 2 * B + B) + jn)
        down = down + jnp.where(own, d, 0.0)
    sfg = jnp.sum(jnp.where(inst > 0, (seed - down) ** 2, 0.0))
    sacc = jnp.where(io == 0, sfg, 0.0)

    @pl.when(k == 0)
    def _():
        sfg_ref[0] = sacc

    @pl.when(k != 0)
    def _():
        sfg_ref[0] = sfg_ref[0] + sacc


def _pass2(scal, prediction, instances, bo, nb):
    rows = 64
    nk = H // rows
    return pl.pallas_call(
        functools.partial(_pass2_body, bo),
        grid=(nb, nk),
        in_specs=[
            pl.BlockSpec(memory_space=pltpu.SMEM),
            pl.BlockSpec((1, 4, rows, W), lambda b, k: (b + bo, 0, k, 0)),
            pl.BlockSpec((1, rows, W), lambda b, k: (b + bo, k, 0)),
        ],
        out_specs=[
            pl.BlockSpec((1, NI, rows, W), lambda b, k: (b, 0, k, 0)),
            pl.BlockSpec((1, 1, 128), lambda b, k: (b, 0, 0)),
        ],
        out_shape=[
            jax.ShapeDtypeStruct((nb, NI, H, W), jnp.int32),
            jax.ShapeDtypeStruct((nb, 1, 128), jnp.float32),
        ],
        interpret=_INTERPRET,
    )(scal, prediction, instances)


def _pass3_body(*refs):
    nsplit = (len(refs) - 2) // 2
    parts_refs = refs[:nsplit]
    sums_ref = refs[nsplit]
    sfg_refs = refs[nsplit + 1:2 * nsplit + 1]
    out_ref = refs[2 * nsplit + 1]
    per = NB // nsplit
    iar = lax.broadcasted_iota(jnp.int32, (B, B), 0)
    iac = lax.broadcasted_iota(jnp.int32, (B, B), 1)
    M = (iar >= iac).astype(jnp.float32)
    total = jnp.float32(0.0)
    for b in range(NB):
        parts_ref = parts_refs[b // per]
        sfg_ref = sfg_refs[b // per]
        bl = b % per
        tb = jnp.sum(parts_ref[bl], axis=0)
        suf = jnp.dot(tb, M, preferred_element_type=jnp.float32)
        inst_loss = jnp.float32(0.0)
        var_loss = jnp.float32(0.0)
        obj = jnp.float32(0.0)
        seed_fg = sfg_ref[bl, 0]
        for i in range(NI):
            G = sums_ref[b, 0, i]
            pres = (G > 0.0).astype(jnp.float32)
            Gs = jnp.maximum(G, 1.0)
            C = suf[2 * i:2 * i + 1]
            Nn = suf[2 * i + 1:2 * i + 2]
            Nt = C + Nn
            J = 1.0 - (G - C) / jnp.maximum(G + Nt - C, 1.0)
            lov = (2.0 / B) * (jnp.sum(J) - 0.5 * J[0, 0])
            inst_loss = inst_loss + pres * lov
            ss = sums_ref[b, 3, i]
            ss2 = sums_ref[b, 4, i]
            mu = ss / Gs
            var_loss = var_loss + pres * (ss2 / Gs - mu * mu)
            obj = obj + pres
        denom = jnp.maximum(obj, 1.0)
        bg = sums_ref[b, 5, 0]
        seed_loss = (bg + seed_fg) / jnp.float32(NPIX)
        total = total + inst_loss / denom + 10.0 * var_loss / denom + seed_loss
    out_ref[0, 0] = total / NB


def _pass3(parts_list, sums, sfg_list):
    n = len(parts_list)
    return pl.pallas_call(
        _pass3_body,
        in_specs=(
            [pl.BlockSpec(memory_space=pltpu.VMEM)] * n
            + [pl.BlockSpec(memory_space=pltpu.SMEM)] * (n + 1)
        ),
        out_specs=pl.BlockSpec(memory_space=pltpu.SMEM),
        out_shape=jax.ShapeDtypeStruct((1, 1), jnp.float32),
        interpret=_INTERPRET,
    )(*parts_list, sums, *sfg_list)


NW = 32
CHROWS = 16
CHUNK = CHROWS * W


def _sc_hist_body(nbatch, idx_hbm, out_hbm, buf0, buf1, table, sem0, sem1):
    slots = NW // nbatch
    rows_per_slot = H // slots
    nch = NI * rows_per_slot // CHROWS
    cid = lax.axis_index("c")
    sid = lax.axis_index("s")
    wid = sid * 2 + cid
    batch = wid // slots
    slot = wid - batch * slots
    base_row = slot * rows_per_slot

    zeros = jnp.zeros((16,), jnp.float32)
    ones = jnp.ones((16,), jnp.float32)

    @plsc.parallel_loop(0, TBL // 16, unroll=8)
    def _(j):
        table[pl.ds(j * 16, 16)] = zeros

    def start(c, buf, sem):
        i = c // (rows_per_slot // CHROWS)
        rb = c - i * (rows_per_slot // CHROWS)
        pltpu.async_copy(
            idx_hbm.at[batch, i, pl.ds(base_row + rb * CHROWS, CHROWS)],
            buf, sem)

    def wait(buf, sem):
        pltpu.make_async_copy(
            idx_hbm.at[batch, 0, pl.ds(0, CHROWS)], buf, sem).wait()

    def process(buf):
        @plsc.parallel_loop(0, CHUNK // 16, unroll=8)
        def _(j):
            v = buf[j >> 5, pl.ds((j & 31) * 16, 16)]
            plsc.addupdate_scatter(table, [v], ones)

    start(0, buf0, sem0)

    def pair_body(p, carry):
        c0 = p * 2
        start(c0 + 1, buf1, sem1)
        wait(buf0, sem0)
        process(buf0)

        @pl.when(c0 + 2 < nch)
        def _():
            start(c0 + 2, buf0, sem0)
        wait(buf1, sem1)
        process(buf1)
        return carry
    lax.fori_loop(0, nch // 2, pair_body, 0)
    pltpu.sync_copy(table, out_hbm.at[wid])


def _sc_hist(idx_half, nbatch):
    mesh = plsc.VectorSubcoreMesh(core_axis_name="c", subcore_axis_name="s")
    f = functools.partial(
        pl.kernel,
        mesh=mesh,
        compiler_params=pltpu.CompilerParams(needs_layout_passes=False),
        out_type=jax.ShapeDtypeStruct((NW, TBL), jnp.float32),
        scratch_types=[
            pltpu.VMEM((CHROWS, W), jnp.int32),
            pltpu.VMEM((CHROWS, W), jnp.int32),
            pltpu.VMEM((TBL,), jnp.float32),
            pltpu.SemaphoreType.DMA,
            pltpu.SemaphoreType.DMA,
        ],
    )(functools.partial(_sc_hist_body, nbatch))
    return f(idx_half)


def kernel(prediction, instances, labels):
    sums = _pass1(prediction, instances, labels)
    nsplit = 2
    per = NB // nsplit
    parts_list, sfg_list = [], []
    for g in range(nsplit):
        idxg, sfgg = _pass2(sums, prediction, instances, g * per, per)
        parts_list.append(
            _sc_hist(idxg, per).reshape(per, NW // per, NPLANE, B))
        sfg_list.append(sfgg[:, 0, :1])

    out = _pass3(parts_list, sums, sfg_list)
    return out.reshape(())


# speedup vs baseline: 1.1885x; 1.0383x over previous
import functools

import jax
import jax.numpy as jnp
from jax import lax
from jax.experimental import pallas as pl
from jax.experimental.pallas import tpu as pltpu
from jax.experimental.pallas import tpu_sc as plsc

HX = 2.0 / 2047.0
HY = 1.0 / 1023.0
H = W = 512
NPIX = H * W
NI = 7
NB = 4
B = 2048
NPLANE = 2 * NI
TBL = NPLANE * B

_INTERPRET = False
_DIAG = 0


def _pass1_body(pred_ref, inst_ref, lab_ref, out_ref):
    r = pl.program_id(1)
    sigma = pred_ref[0, 0]
    seed = jax.nn.sigmoid(pred_ref[0, 1])
    inst = inst_ref[0]
    lab = lab_ref[0]
    rows = sigma.shape[0]
    row0 = (r * rows).astype(jnp.float32)
    xm = lax.broadcasted_iota(jnp.int32, sigma.shape, 1).astype(jnp.float32) * HX
    ym = (lax.broadcasted_iota(jnp.int32, sigma.shape, 0).astype(jnp.float32) + row0) * HY

    io = lax.broadcasted_iota(jnp.int32, (1, 128), 1)
    bg = jnp.sum(jnp.where(lab == 0, seed * seed, 0.0))
    zero = jnp.zeros((1, 128), jnp.float32)
    cntv, sxv, syv, ssv, ss2v = zero, zero, zero, zero, zero
    bgv = jnp.where(io == 0, bg, 0.0)
    for i in range(NI):
        mf = (inst == (i + 1)).astype(jnp.float32)
        sel = (io == i)
        cntv = cntv + jnp.where(sel, jnp.sum(mf), 0.0)
        sxv = sxv + jnp.where(sel, jnp.sum(mf * xm), 0.0)
        syv = syv + jnp.where(sel, jnp.sum(mf * ym), 0.0)
        ssv = ssv + jnp.where(sel, jnp.sum(mf * sigma), 0.0)
        ss2v = ss2v + jnp.where(sel, jnp.sum(mf * sigma * sigma), 0.0)
    acc = jnp.concatenate([cntv, sxv, syv, ssv, ss2v, bgv], axis=0)

    @pl.when(r == 0)
    def _():
        out_ref[0] = acc

    @pl.when(r != 0)
    def _():
        out_ref[0] = out_ref[0] + acc


def _pass1(prediction, instances, labels):
    rows = 512
    nr = H // rows
    return pl.pallas_call(
        _pass1_body,
        grid=(NB, nr),
        in_specs=[
            pl.BlockSpec((1, 2, rows, W), lambda b, r: (b, 1, r, 0)),
            pl.BlockSpec((1, rows, W), lambda b, r: (b, r, 0)),
            pl.BlockSpec((1, rows, W), lambda b, r: (b, r, 0)),
        ],
        out_specs=pl.BlockSpec((1, 6, 128), lambda b, r: (b, 0, 0)),
        out_shape=jax.ShapeDtypeStruct((NB, 6, 128), jnp.float32),
        interpret=_INTERPRET,
    )(prediction, instances, labels)


def _pass2_body(bo, scal_ref, pred_ref, inst_ref, idx_ref, sfg_ref):
    b = pl.program_id(0) + bo
    k = pl.program_id(1)
    p = pred_ref[0]
    rows = p.shape[1]
    row0 = (k * rows).astype(jnp.float32)
    xm = lax.broadcasted_iota(jnp.int32, (rows, W), 1).astype(jnp.float32) * HX
    ym = (lax.broadcasted_iota(jnp.int32, (rows, W), 0).astype(jnp.float32) + row0) * HY
    ex = jnp.tanh(p[0]) + xm
    ey = jnp.tanh(p[1]) + ym
    sig = p[2]
    seed = jax.nn.sigmoid(p[3])
    inst = inst_ref[0]

    io = lax.broadcasted_iota(jnp.int32, (1, 128), 1)
    bf = jnp.float32(B)
    r2p = ex * ex + ey * ey
    down = jnp.zeros_like(ex)
    for i in range(NI):
        safe = jnp.maximum(scal_ref[b, 0, i], 1.0)
        cx = scal_ref[b, 1, i] / safe
        cy = scal_ref[b, 2, i] / safe
        s = jnp.exp(10.0 * scal_ref[b, 3, i] / safe)
        bx = -2.0 * s * cx
        by = -2.0 * s * cy
        c0 = s * (cx * cx + cy * cy)
        u = s * r2p + bx * ex + by * ey + c0
        d = jnp.exp(-u)
        own = inst == (i + 1)
        jn = jnp.minimum((bf * d).astype(jnp.int32), B - 1)
        idx_ref[0, i] = jnp.where(own, (i * 2 * B + B - 1) - jn,
                                  (i * 2 * B + B) + jn)
        down = down + jnp.where(own, d, 0.0)
    sfg = jnp.sum(jnp.where(inst > 0, (seed - down) ** 2, 0.0))
    sacc = jnp.where(io == 0, sfg, 0.0)

    @pl.when(k == 0)
    def _():
        sfg_ref[0] = sacc

    @pl.when(k != 0)
    def _():
        sfg_ref[0] = sfg_ref[0] + sacc


def _pass2(scal, prediction, instances, bo, nb):
    rows = 128
    nk = H // rows
    return pl.pallas_call(
        functools.partial(_pass2_body, bo),
        grid=(nb, nk),
        in_specs=[
            pl.BlockSpec(memory_space=pltpu.SMEM),
            pl.BlockSpec((1, 4, rows, W), lambda b, k: (b + bo, 0, k, 0)),
            pl.BlockSpec((1, rows, W), lambda b, k: (b + bo, k, 0)),
        ],
        out_specs=[
            pl.BlockSpec((1, NI, rows, W), lambda b, k: (b, 0, k, 0)),
            pl.BlockSpec((1, 1, 128), lambda b, k: (b, 0, 0)),
        ],
        out_shape=[
            jax.ShapeDtypeStruct((nb, NI, H, W), jnp.int32),
            jax.ShapeDtypeStruct((nb, 1, 128), jnp.float32),
        ],
        interpret=_INTERPRET,
    )(scal, prediction, instances)


def _pass3_body(*refs):
    nsplit = (len(refs) - 2) // 2
    parts_refs = refs[:nsplit]
    sums_ref = refs[nsplit]
    sfg_refs = refs[nsplit + 1:2 * nsplit + 1]
    out_ref = refs[2 * nsplit + 1]
    per = NB // nsplit
    iar = lax.broadcasted_iota(jnp.int32, (B, B), 0)
    iac = lax.broadcasted_iota(jnp.int32, (B, B), 1)
    M = (iar >= iac).astype(jnp.float32)
    total = jnp.float32(0.0)
    for b in range(NB):
        parts_ref = parts_refs[b // per]
        sfg_ref = sfg_refs[b // per]
        bl = b % per
        tb = jnp.sum(parts_ref[bl], axis=0)
        suf = jnp.dot(tb, M, preferred_element_type=jnp.float32)
        inst_loss = jnp.float32(0.0)
        var_loss = jnp.float32(0.0)
        obj = jnp.float32(0.0)
        seed_fg = sfg_ref[bl, 0]
        for i in range(NI):
            G = sums_ref[b, 0, i]
            pres = (G > 0.0).astype(jnp.float32)
            Gs = jnp.maximum(G, 1.0)
            C = suf[2 * i:2 * i + 1]
            Nn = suf[2 * i + 1:2 * i + 2]
            Nt = C + Nn
            J = 1.0 - (G - C) / jnp.maximum(G + Nt - C, 1.0)
            lov = (2.0 / B) * (jnp.sum(J) - 0.5 * J[0, 0])
            inst_loss = inst_loss + pres * lov
            ss = sums_ref[b, 3, i]
            ss2 = sums_ref[b, 4, i]
            mu = ss / Gs
            var_loss = var_loss + pres * (ss2 / Gs - mu * mu)
            obj = obj + pres
        denom = jnp.maximum(obj, 1.0)
        bg = sums_ref[b, 5, 0]
        seed_loss = (bg + seed_fg) / jnp.float32(NPIX)
        total = total + inst_loss / denom + 10.0 * var_loss / denom + seed_loss
    out_ref[0, 0] = total / NB


def _pass3(parts_list, sums, sfg_list):
    n = len(parts_list)
    return pl.pallas_call(
        _pass3_body,
        in_specs=(
            [pl.BlockSpec(memory_space=pltpu.VMEM)] * n
            + [pl.BlockSpec(memory_space=pltpu.SMEM)] * (n + 1)
        ),
        out_specs=pl.BlockSpec(memory_space=pltpu.SMEM),
        out_shape=jax.ShapeDtypeStruct((1, 1), jnp.float32),
        interpret=_INTERPRET,
    )(*parts_list, sums, *sfg_list)


NW = 32
CHROWS = 16
CHUNK = CHROWS * W


def _sc_hist_body(nbatch, idx_hbm, out_hbm, buf0, buf1, table, sem0, sem1):
    slots = NW // nbatch
    rows_per_slot = H // slots
    nch = NI * rows_per_slot // CHROWS
    cid = lax.axis_index("c")
    sid = lax.axis_index("s")
    wid = sid * 2 + cid
    batch = wid // slots
    slot = wid - batch * slots
    base_row = slot * rows_per_slot

    zeros = jnp.zeros((16,), jnp.float32)
    ones = jnp.ones((16,), jnp.float32)

    @plsc.parallel_loop(0, TBL // 16, unroll=8)
    def _(j):
        table[pl.ds(j * 16, 16)] = zeros

    def start(c, buf, sem):
        i = c // (rows_per_slot // CHROWS)
        rb = c - i * (rows_per_slot // CHROWS)
        pltpu.async_copy(
            idx_hbm.at[batch, i, pl.ds(base_row + rb * CHROWS, CHROWS)],
            buf, sem)

    def wait(buf, sem):
        pltpu.make_async_copy(
            idx_hbm.at[batch, 0, pl.ds(0, CHROWS)], buf, sem).wait()

    def process(buf):
        @plsc.parallel_loop(0, CHUNK // 16, unroll=16)
        def _(j):
            v = buf[j >> 5, pl.ds((j & 31) * 16, 16)]
            plsc.addupdate_scatter(table, [v], ones)

    start(0, buf0, sem0)

    def pair_body(p, carry):
        c0 = p * 2
        start(c0 + 1, buf1, sem1)
        wait(buf0, sem0)
        process(buf0)

        @pl.when(c0 + 2 < nch)
        def _():
            start(c0 + 2, buf0, sem0)
        wait(buf1, sem1)
        process(buf1)
        return carry
    lax.fori_loop(0, nch // 2, pair_body, 0)
    pltpu.sync_copy(table, out_hbm.at[wid])


def _sc_hist(idx_half, nbatch):
    mesh = plsc.VectorSubcoreMesh(core_axis_name="c", subcore_axis_name="s")
    f = functools.partial(
        pl.kernel,
        mesh=mesh,
        compiler_params=pltpu.CompilerParams(needs_layout_passes=False),
        out_type=jax.ShapeDtypeStruct((NW, TBL), jnp.float32),
        scratch_types=[
            pltpu.VMEM((CHROWS, W), jnp.int32),
            pltpu.VMEM((CHROWS, W), jnp.int32),
            pltpu.VMEM((TBL,), jnp.float32),
            pltpu.SemaphoreType.DMA,
            pltpu.SemaphoreType.DMA,
        ],
    )(functools.partial(_sc_hist_body, nbatch))
    return f(idx_half)


def kernel(prediction, instances, labels):
    sums = _pass1(prediction, instances, labels)
    nsplit = 2
    per = NB // nsplit
    parts_list, sfg_list = [], []
    for g in range(nsplit):
        idxg, sfgg = _pass2(sums, prediction, instances, g * per, per)
        parts_list.append(
            _sc_hist(idxg, per).reshape(per, NW // per, NPLANE, B))
        sfg_list.append(sfgg[:, 0, :1])

    out = _pass3(parts_list, sums, sfg_list)
    return out.reshape(())


# speedup vs baseline: 1.3188x; 1.1096x over previous
import functools

import jax
import jax.numpy as jnp
from jax import lax
from jax.experimental import pallas as pl
from jax.experimental.pallas import tpu as pltpu
from jax.experimental.pallas import tpu_sc as plsc

HX = 2.0 / 2047.0
HY = 1.0 / 1023.0
H = W = 512
NPIX = H * W
NI = 7
NB = 4
B = 1024
NPLANE = 2 * NI
TBL = NPLANE * B

_INTERPRET = False
_DIAG = 0


def _pass1_body(pred_ref, inst_ref, lab_ref, out_ref):
    r = pl.program_id(1)
    sigma = pred_ref[0, 0]
    seed = jax.nn.sigmoid(pred_ref[0, 1])
    inst = inst_ref[0]
    lab = lab_ref[0]
    rows = sigma.shape[0]
    row0 = (r * rows).astype(jnp.float32)
    xm = lax.broadcasted_iota(jnp.int32, sigma.shape, 1).astype(jnp.float32) * HX
    ym = (lax.broadcasted_iota(jnp.int32, sigma.shape, 0).astype(jnp.float32) + row0) * HY

    io = lax.broadcasted_iota(jnp.int32, (1, 128), 1)
    bg = jnp.sum(jnp.where(lab == 0, seed * seed, 0.0))
    zero = jnp.zeros((1, 128), jnp.float32)
    cntv, sxv, syv, ssv, ss2v = zero, zero, zero, zero, zero
    bgv = jnp.where(io == 0, bg, 0.0)
    for i in range(NI):
        mf = (inst == (i + 1)).astype(jnp.float32)
        sel = (io == i)
        cntv = cntv + jnp.where(sel, jnp.sum(mf), 0.0)
        sxv = sxv + jnp.where(sel, jnp.sum(mf * xm), 0.0)
        syv = syv + jnp.where(sel, jnp.sum(mf * ym), 0.0)
        ssv = ssv + jnp.where(sel, jnp.sum(mf * sigma), 0.0)
        ss2v = ss2v + jnp.where(sel, jnp.sum(mf * sigma * sigma), 0.0)
    acc = jnp.concatenate([cntv, sxv, syv, ssv, ss2v, bgv], axis=0)

    @pl.when(r == 0)
    def _():
        out_ref[0] = acc

    @pl.when(r != 0)
    def _():
        out_ref[0] = out_ref[0] + acc


def _pass1(prediction, instances, labels):
    rows = 512
    nr = H // rows
    return pl.pallas_call(
        _pass1_body,
        grid=(NB, nr),
        in_specs=[
            pl.BlockSpec((1, 2, rows, W), lambda b, r: (b, 1, r, 0)),
            pl.BlockSpec((1, rows, W), lambda b, r: (b, r, 0)),
            pl.BlockSpec((1, rows, W), lambda b, r: (b, r, 0)),
        ],
        out_specs=pl.BlockSpec((1, 6, 128), lambda b, r: (b, 0, 0)),
        out_shape=jax.ShapeDtypeStruct((NB, 6, 128), jnp.float32),
        interpret=_INTERPRET,
    )(prediction, instances, labels)


def _pass2_body(bo, scal_ref, pred_ref, inst_ref, idx_ref, sfg_ref):
    b = pl.program_id(0) + bo
    k = pl.program_id(1)
    p = pred_ref[0]
    rows = p.shape[1]
    row0 = (k * rows).astype(jnp.float32)
    xm = lax.broadcasted_iota(jnp.int32, (rows, W), 1).astype(jnp.float32) * HX
    ym = (lax.broadcasted_iota(jnp.int32, (rows, W), 0).astype(jnp.float32) + row0) * HY
    ex = jnp.tanh(p[0]) + xm
    ey = jnp.tanh(p[1]) + ym
    sig = p[2]
    seed = jax.nn.sigmoid(p[3])
    inst = inst_ref[0]

    io = lax.broadcasted_iota(jnp.int32, (1, 128), 1)
    bf = jnp.float32(B)
    r2p = ex * ex + ey * ey
    down = jnp.zeros_like(ex)
    for i in range(NI):
        safe = jnp.maximum(scal_ref[b, 0, i], 1.0)
        cx = scal_ref[b, 1, i] / safe
        cy = scal_ref[b, 2, i] / safe
        s = jnp.exp(10.0 * scal_ref[b, 3, i] / safe)
        bx = -2.0 * s * cx
        by = -2.0 * s * cy
        c0 = s * (cx * cx + cy * cy)
        u = s * r2p + bx * ex + by * ey + c0
        d = jnp.exp(-u)
        own = inst == (i + 1)
        jn = jnp.minimum((bf * d).astype(jnp.int32), B - 1)
        idx_ref[0, i] = jnp.where(own, (i * 2 * B + B - 1) - jn,
                                  (i * 2 * B + B) + jn)
        down = down + jnp.where(own, d, 0.0)
    sfg = jnp.sum(jnp.where(inst > 0, (seed - down) ** 2, 0.0))
    sacc = jnp.where(io == 0, sfg, 0.0)

    @pl.when(k == 0)
    def _():
        sfg_ref[0] = sacc

    @pl.when(k != 0)
    def _():
        sfg_ref[0] = sfg_ref[0] + sacc


def _pass2(scal, prediction, instances, bo, nb):
    rows = 128
    nk = H // rows
    return pl.pallas_call(
        functools.partial(_pass2_body, bo),
        grid=(nb, nk),
        in_specs=[
            pl.BlockSpec(memory_space=pltpu.SMEM),
            pl.BlockSpec((1, 4, rows, W), lambda b, k: (b + bo, 0, k, 0)),
            pl.BlockSpec((1, rows, W), lambda b, k: (b + bo, k, 0)),
        ],
        out_specs=[
            pl.BlockSpec((1, NI, rows, W), lambda b, k: (b, 0, k, 0)),
            pl.BlockSpec((1, 1, 128), lambda b, k: (b, 0, 0)),
        ],
        out_shape=[
            jax.ShapeDtypeStruct((nb, NI, H, W), jnp.int32),
            jax.ShapeDtypeStruct((nb, 1, 128), jnp.float32),
        ],
        interpret=_INTERPRET,
    )(scal, prediction, instances)


def _pass3_body(*refs):
    nsplit = (len(refs) - 2) // 2
    parts_refs = refs[:nsplit]
    sums_ref = refs[nsplit]
    sfg_refs = refs[nsplit + 1:2 * nsplit + 1]
    out_ref = refs[2 * nsplit + 1]
    per = NB // nsplit
    slots = NW // per
    iar = lax.broadcasted_iota(jnp.int32, (B, B), 0)
    iac = lax.broadcasted_iota(jnp.int32, (B, B), 1)
    M = (iar >= iac).astype(jnp.float32)
    total = jnp.float32(0.0)
    for b in range(NB):
        parts_ref = parts_refs[b // per]
        sfg_ref = sfg_refs[b // per]
        bl = b % per
        tb1 = jnp.sum(parts_ref[bl * slots:(bl + 1) * slots], axis=0,
                      keepdims=True)
        tb = jnp.concatenate(
            [tb1[:, p * B:(p + 1) * B] for p in range(NPLANE)], axis=0)
        suf = jnp.dot(tb, M, preferred_element_type=jnp.float32)
        inst_loss = jnp.float32(0.0)
        var_loss = jnp.float32(0.0)
        obj = jnp.float32(0.0)
        seed_fg = sfg_ref[bl, 0]
        for i in range(NI):
            G = sums_ref[b, 0, i]
            pres = (G > 0.0).astype(jnp.float32)
            Gs = jnp.maximum(G, 1.0)
            C = suf[2 * i:2 * i + 1]
            Nn = suf[2 * i + 1:2 * i + 2]
            Nt = C + Nn
            J = 1.0 - (G - C) / jnp.maximum(G + Nt - C, 1.0)
            lov = (2.0 / B) * (jnp.sum(J) - 0.5 * J[0, 0])
            inst_loss = inst_loss + pres * lov
            ss = sums_ref[b, 3, i]
            ss2 = sums_ref[b, 4, i]
            mu = ss / Gs
            var_loss = var_loss + pres * (ss2 / Gs - mu * mu)
            obj = obj + pres
        denom = jnp.maximum(obj, 1.0)
        bg = sums_ref[b, 5, 0]
        seed_loss = (bg + seed_fg) / jnp.float32(NPIX)
        total = total + inst_loss / denom + 10.0 * var_loss / denom + seed_loss
    out_ref[0, 0] = total / NB


def _pass3(parts_list, sums, sfg_list):
    n = len(parts_list)
    return pl.pallas_call(
        _pass3_body,
        in_specs=(
            [pl.BlockSpec(memory_space=pltpu.VMEM)] * n
            + [pl.BlockSpec(memory_space=pltpu.SMEM)] * (n + 1)
        ),
        out_specs=pl.BlockSpec(memory_space=pltpu.SMEM),
        out_shape=jax.ShapeDtypeStruct((1, 1), jnp.float32),
        interpret=_INTERPRET,
    )(*parts_list, sums, *sfg_list)


NW = 32
CHROWS = 16
CHUNK = CHROWS * W


def _sc_hist_body(nbatch, idx_hbm, out_hbm, buf0, buf1, table, sem0, sem1):
    slots = NW // nbatch
    rows_per_slot = H // slots
    nch = NI * rows_per_slot // CHROWS
    cid = lax.axis_index("c")
    sid = lax.axis_index("s")
    wid = sid * 2 + cid
    batch = wid // slots
    slot = wid - batch * slots
    base_row = slot * rows_per_slot

    zeros = jnp.zeros((16,), jnp.float32)
    ones = jnp.ones((16,), jnp.float32)

    @plsc.parallel_loop(0, TBL // 16, unroll=8)
    def _(j):
        table[pl.ds(j * 16, 16)] = zeros

    def start(c, buf, sem):
        i = c // (rows_per_slot // CHROWS)
        rb = c - i * (rows_per_slot // CHROWS)
        pltpu.async_copy(
            idx_hbm.at[batch, i, pl.ds(base_row + rb * CHROWS, CHROWS)],
            buf, sem)

    def wait(buf, sem):
        pltpu.make_async_copy(
            idx_hbm.at[batch, 0, pl.ds(0, CHROWS)], buf, sem).wait()

    def process(buf):
        @plsc.parallel_loop(0, CHUNK // 16, unroll=16)
        def _(j):
            v = buf[j >> 5, pl.ds((j & 31) * 16, 16)]
            plsc.addupdate_scatter(table, [v], ones)

    start(0, buf0, sem0)

    def pair_body(p, carry):
        c0 = p * 2
        start(c0 + 1, buf1, sem1)
        wait(buf0, sem0)
        process(buf0)

        @pl.when(c0 + 2 < nch)
        def _():
            start(c0 + 2, buf0, sem0)
        wait(buf1, sem1)
        process(buf1)
        return carry
    lax.fori_loop(0, nch // 2, pair_body, 0)
    pltpu.sync_copy(table, out_hbm.at[wid])


def _sc_hist(idx_half, nbatch):
    mesh = plsc.VectorSubcoreMesh(core_axis_name="c", subcore_axis_name="s")
    f = functools.partial(
        pl.kernel,
        mesh=mesh,
        compiler_params=pltpu.CompilerParams(needs_layout_passes=False),
        out_type=jax.ShapeDtypeStruct((NW, TBL), jnp.float32),
        scratch_types=[
            pltpu.VMEM((CHROWS, W), jnp.int32),
            pltpu.VMEM((CHROWS, W), jnp.int32),
            pltpu.VMEM((TBL,), jnp.float32),
            pltpu.SemaphoreType.DMA,
            pltpu.SemaphoreType.DMA,
        ],
    )(functools.partial(_sc_hist_body, nbatch))
    return f(idx_half)


def kernel(prediction, instances, labels):
    sums = _pass1(prediction, instances, labels)
    nsplit = 2
    per = NB // nsplit
    parts_list, sfg_list = [], []
    for g in range(nsplit):
        idxg, sfgg = _pass2(sums, prediction, instances, g * per, per)
        parts_list.append(_sc_hist(idxg, per))
        sfg_list.append(sfgg[:, 0, :1])

    out = _pass3(parts_list, sums, sfg_list)
    return out.reshape(())


# speedup vs baseline: 1.3690x; 1.0381x over previous
import functools

import jax
import jax.numpy as jnp
from jax import lax
from jax.experimental import pallas as pl
from jax.experimental.pallas import tpu as pltpu
from jax.experimental.pallas import tpu_sc as plsc

HX = 2.0 / 2047.0
HY = 1.0 / 1023.0
H = W = 512
NPIX = H * W
NI = 7
NB = 4
B = 1024
NPLANE = 2 * NI
TBL = NPLANE * B

_INTERPRET = False
_DIAG = 0


def _pass1_body(pred_ref, inst_ref, lab_ref, out_ref):
    r = pl.program_id(1)
    sigma = pred_ref[0, 0]
    seed = jax.nn.sigmoid(pred_ref[0, 1])
    inst = inst_ref[0]
    lab = lab_ref[0]
    rows = sigma.shape[0]
    row0 = (r * rows).astype(jnp.float32)
    xm = lax.broadcasted_iota(jnp.int32, sigma.shape, 1).astype(jnp.float32) * HX
    ym = (lax.broadcasted_iota(jnp.int32, sigma.shape, 0).astype(jnp.float32) + row0) * HY

    io = lax.broadcasted_iota(jnp.int32, (1, 128), 1)
    bg = jnp.sum(jnp.where(lab == 0, seed * seed, 0.0))
    zero = jnp.zeros((1, 128), jnp.float32)
    cntv, sxv, syv, ssv, ss2v = zero, zero, zero, zero, zero
    bgv = jnp.where(io == 0, bg, 0.0)
    for i in range(NI):
        mf = (inst == (i + 1)).astype(jnp.float32)
        sel = (io == i)
        cntv = cntv + jnp.where(sel, jnp.sum(mf), 0.0)
        sxv = sxv + jnp.where(sel, jnp.sum(mf * xm), 0.0)
        syv = syv + jnp.where(sel, jnp.sum(mf * ym), 0.0)
        ssv = ssv + jnp.where(sel, jnp.sum(mf * sigma), 0.0)
        ss2v = ss2v + jnp.where(sel, jnp.sum(mf * sigma * sigma), 0.0)
    acc = jnp.concatenate([cntv, sxv, syv, ssv, ss2v, bgv], axis=0)

    @pl.when(r == 0)
    def _():
        out_ref[0] = acc

    @pl.when(r != 0)
    def _():
        out_ref[0] = out_ref[0] + acc


def _pass1(prediction, instances, labels):
    rows = 512
    nr = H // rows
    return pl.pallas_call(
        _pass1_body,
        grid=(NB, nr),
        in_specs=[
            pl.BlockSpec((1, 2, rows, W), lambda b, r: (b, 1, r, 0)),
            pl.BlockSpec((1, rows, W), lambda b, r: (b, r, 0)),
            pl.BlockSpec((1, rows, W), lambda b, r: (b, r, 0)),
        ],
        out_specs=pl.BlockSpec((1, 6, 128), lambda b, r: (b, 0, 0)),
        out_shape=jax.ShapeDtypeStruct((NB, 6, 128), jnp.float32),
        interpret=_INTERPRET,
    )(prediction, instances, labels)


def _pass2_body(bo, scal_ref, pred_ref, inst_ref, idx_ref, sfg_ref):
    b = pl.program_id(0) + bo
    k = pl.program_id(1)
    p = pred_ref[0]
    rows = p.shape[1]
    row0 = (k * rows).astype(jnp.float32)
    xm = lax.broadcasted_iota(jnp.int32, (rows, W), 1).astype(jnp.float32) * HX
    ym = (lax.broadcasted_iota(jnp.int32, (rows, W), 0).astype(jnp.float32) + row0) * HY
    ex = jnp.tanh(p[0]) + xm
    ey = jnp.tanh(p[1]) + ym
    sig = p[2]
    seed = jax.nn.sigmoid(p[3])
    inst = inst_ref[0]

    io = lax.broadcasted_iota(jnp.int32, (1, 128), 1)
    bf = jnp.float32(B)
    r2p = ex * ex + ey * ey
    down = jnp.zeros_like(ex)
    for i in range(NI):
        safe = jnp.maximum(scal_ref[b, 0, i], 1.0)
        cx = scal_ref[b, 1, i] / safe
        cy = scal_ref[b, 2, i] / safe
        s = jnp.exp(10.0 * scal_ref[b, 3, i] / safe)
        bx = -2.0 * s * cx
        by = -2.0 * s * cy
        c0 = s * (cx * cx + cy * cy)
        u = s * r2p + bx * ex + by * ey + c0
        d = jnp.exp(-u)
        own = inst == (i + 1)
        jn = jnp.minimum((bf * d).astype(jnp.int32), B - 1)
        idx_ref[0, i] = jnp.where(own, (i * 2 * B + B - 1) - jn,
                                  (i * 2 * B + B) + jn).astype(jnp.int16)
        down = down + jnp.where(own, d, 0.0)
    sfg = jnp.sum(jnp.where(inst > 0, (seed - down) ** 2, 0.0))
    sacc = jnp.where(io == 0, sfg, 0.0)

    @pl.when(k == 0)
    def _():
        sfg_ref[0] = sacc

    @pl.when(k != 0)
    def _():
        sfg_ref[0] = sfg_ref[0] + sacc


def _pass2(scal, prediction, instances, bo, nb):
    rows = 128
    nk = H // rows
    return pl.pallas_call(
        functools.partial(_pass2_body, bo),
        grid=(nb, nk),
        in_specs=[
            pl.BlockSpec(memory_space=pltpu.SMEM),
            pl.BlockSpec((1, 4, rows, W), lambda b, k: (b + bo, 0, k, 0)),
            pl.BlockSpec((1, rows, W), lambda b, k: (b + bo, k, 0)),
        ],
        out_specs=[
            pl.BlockSpec((1, NI, rows, W), lambda b, k: (b, 0, k, 0)),
            pl.BlockSpec((1, 1, 128), lambda b, k: (b, 0, 0)),
        ],
        out_shape=[
            jax.ShapeDtypeStruct((nb, NI, H, W), jnp.int16),
            jax.ShapeDtypeStruct((nb, 1, 128), jnp.float32),
        ],
        interpret=_INTERPRET,
    )(scal, prediction, instances)


def _pass3_body(*refs):
    nsplit = (len(refs) - 2) // 2
    parts_refs = refs[:nsplit]
    sums_ref = refs[nsplit]
    sfg_refs = refs[nsplit + 1:2 * nsplit + 1]
    out_ref = refs[2 * nsplit + 1]
    per = NB // nsplit
    slots = NW // per
    iar = lax.broadcasted_iota(jnp.int32, (B, B), 0)
    iac = lax.broadcasted_iota(jnp.int32, (B, B), 1)
    M = (iar >= iac).astype(jnp.float32)
    total = jnp.float32(0.0)
    for b in range(NB):
        parts_ref = parts_refs[b // per]
        sfg_ref = sfg_refs[b // per]
        bl = b % per
        tb1 = jnp.sum(parts_ref[bl * slots:(bl + 1) * slots], axis=0,
                      keepdims=True)
        tb = jnp.concatenate(
            [tb1[:, p * B:(p + 1) * B] for p in range(NPLANE)], axis=0)
        suf = jnp.dot(tb, M, preferred_element_type=jnp.float32)
        inst_loss = jnp.float32(0.0)
        var_loss = jnp.float32(0.0)
        obj = jnp.float32(0.0)
        seed_fg = sfg_ref[bl, 0]
        for i in range(NI):
            G = sums_ref[b, 0, i]
            pres = (G > 0.0).astype(jnp.float32)
            Gs = jnp.maximum(G, 1.0)
            C = suf[2 * i:2 * i + 1]
            Nn = suf[2 * i + 1:2 * i + 2]
            Nt = C + Nn
            J = 1.0 - (G - C) / jnp.maximum(G + Nt - C, 1.0)
            lov = (2.0 / B) * (jnp.sum(J) - 0.5 * J[0, 0])
            inst_loss = inst_loss + pres * lov
            ss = sums_ref[b, 3, i]
            ss2 = sums_ref[b, 4, i]
            mu = ss / Gs
            var_loss = var_loss + pres * (ss2 / Gs - mu * mu)
            obj = obj + pres
        denom = jnp.maximum(obj, 1.0)
        bg = sums_ref[b, 5, 0]
        seed_loss = (bg + seed_fg) / jnp.float32(NPIX)
        total = total + inst_loss / denom + 10.0 * var_loss / denom + seed_loss
    out_ref[0, 0] = total / NB


def _pass3(parts_list, sums, sfg_list):
    n = len(parts_list)
    return pl.pallas_call(
        _pass3_body,
        in_specs=(
            [pl.BlockSpec(memory_space=pltpu.VMEM)] * n
            + [pl.BlockSpec(memory_space=pltpu.SMEM)] * (n + 1)
        ),
        out_specs=pl.BlockSpec(memory_space=pltpu.SMEM),
        out_shape=jax.ShapeDtypeStruct((1, 1), jnp.float32),
        interpret=_INTERPRET,
    )(*parts_list, sums, *sfg_list)


NW = 32
CHROWS = 16
CHUNK = CHROWS * W


def _sc_hist_body(nbatch, idx_hbm, out_hbm, buf0, buf1, table, sem0, sem1):
    slots = NW // nbatch
    rows_per_slot = H // slots
    nch = NI * rows_per_slot // CHROWS
    cid = lax.axis_index("c")
    sid = lax.axis_index("s")
    wid = sid * 2 + cid
    batch = wid // slots
    slot = wid - batch * slots
    base_row = slot * rows_per_slot

    zeros = jnp.zeros((16,), jnp.float32)
    ones = jnp.ones((16,), jnp.float32)

    @plsc.parallel_loop(0, TBL // 16, unroll=8)
    def _(j):
        table[pl.ds(j * 16, 16)] = zeros

    def start(c, buf, sem):
        i = c // (rows_per_slot // CHROWS)
        rb = c - i * (rows_per_slot // CHROWS)
        pltpu.async_copy(
            idx_hbm.at[batch, i, pl.ds(base_row + rb * CHROWS, CHROWS)],
            buf, sem)

    def wait(buf, sem):
        pltpu.make_async_copy(
            idx_hbm.at[batch, 0, pl.ds(0, CHROWS)], buf, sem).wait()

    def process(buf):
        @plsc.parallel_loop(0, CHUNK // 32, unroll=8)
        def _(j):
            v = buf[j >> 4, pl.ds((j & 15) * 32, 32)]
            a, bb = plsc.unpack(v, format=plsc.PackFormat.INTERLEAVED)
            plsc.addupdate_scatter(table, [a], ones)
            plsc.addupdate_scatter(table, [bb], ones)

    start(0, buf0, sem0)

    def pair_body(p, carry):
        c0 = p * 2
        start(c0 + 1, buf1, sem1)
        wait(buf0, sem0)
        process(buf0)

        @pl.when(c0 + 2 < nch)
        def _():
            start(c0 + 2, buf0, sem0)
        wait(buf1, sem1)
        process(buf1)
        return carry
    lax.fori_loop(0, nch // 2, pair_body, 0)
    pltpu.sync_copy(table, out_hbm.at[wid])


def _sc_hist(idx_half, nbatch):
    mesh = plsc.VectorSubcoreMesh(core_axis_name="c", subcore_axis_name="s")
    f = functools.partial(
        pl.kernel,
        mesh=mesh,
        compiler_params=pltpu.CompilerParams(needs_layout_passes=False),
        out_type=jax.ShapeDtypeStruct((NW, TBL), jnp.float32),
        scratch_types=[
            pltpu.VMEM((CHROWS, W), jnp.int16),
            pltpu.VMEM((CHROWS, W), jnp.int16),
            pltpu.VMEM((TBL,), jnp.float32),
            pltpu.SemaphoreType.DMA,
            pltpu.SemaphoreType.DMA,
        ],
    )(functools.partial(_sc_hist_body, nbatch))
    return f(idx_half)


def kernel(prediction, instances, labels):
    sums = _pass1(prediction, instances, labels)
    nsplit = 2
    per = NB // nsplit
    parts_list, sfg_list = [], []
    for g in range(nsplit):
        idxg, sfgg = _pass2(sums, prediction, instances, g * per, per)
        parts_list.append(_sc_hist(idxg, per))
        sfg_list.append(sfgg[:, 0, :1])

    out = _pass3(parts_list, sums, sfg_list)
    return out.reshape(())


# speedup vs baseline: 1.3713x; 1.0017x over previous
import functools

import jax
import jax.numpy as jnp
from jax import lax
from jax.experimental import pallas as pl
from jax.experimental.pallas import tpu as pltpu
from jax.experimental.pallas import tpu_sc as plsc

HX = 2.0 / 2047.0
HY = 1.0 / 1023.0
H = W = 512
NPIX = H * W
NI = 7
NB = 4
B = 1024
NPLANE = 2 * NI
TBL = NPLANE * B


def _pass1_body(pred_ref, inst_ref, lab_ref, out_ref):
    r = pl.program_id(1)
    sigma = pred_ref[0, 0]
    seed = jax.nn.sigmoid(pred_ref[0, 1])
    inst = inst_ref[0]
    lab = lab_ref[0]
    rows = sigma.shape[0]
    row0 = (r * rows).astype(jnp.float32)
    xm = lax.broadcasted_iota(jnp.int32, sigma.shape, 1).astype(jnp.float32) * HX
    ym = (lax.broadcasted_iota(jnp.int32, sigma.shape, 0).astype(jnp.float32) + row0) * HY

    io = lax.broadcasted_iota(jnp.int32, (1, 128), 1)
    bg = jnp.sum(jnp.where(lab == 0, seed * seed, 0.0))
    zero = jnp.zeros((1, 128), jnp.float32)
    cntv, sxv, syv, ssv, ss2v = zero, zero, zero, zero, zero
    bgv = jnp.where(io == 0, bg, 0.0)
    for i in range(NI):
        mf = (inst == (i + 1)).astype(jnp.float32)
        sel = (io == i)
        cntv = cntv + jnp.where(sel, jnp.sum(mf), 0.0)
        sxv = sxv + jnp.where(sel, jnp.sum(mf * xm), 0.0)
        syv = syv + jnp.where(sel, jnp.sum(mf * ym), 0.0)
        ssv = ssv + jnp.where(sel, jnp.sum(mf * sigma), 0.0)
        ss2v = ss2v + jnp.where(sel, jnp.sum(mf * sigma * sigma), 0.0)
    acc = jnp.concatenate([cntv, sxv, syv, ssv, ss2v, bgv], axis=0)

    @pl.when(r == 0)
    def _():
        out_ref[0] = acc

    @pl.when(r != 0)
    def _():
        out_ref[0] = out_ref[0] + acc


def _pass1(prediction, instances, labels):
    rows = 512
    nr = H // rows
    return pl.pallas_call(
        _pass1_body,
        grid=(NB, nr),
        in_specs=[
            pl.BlockSpec((1, 2, rows, W), lambda b, r: (b, 1, r, 0)),
            pl.BlockSpec((1, rows, W), lambda b, r: (b, r, 0)),
            pl.BlockSpec((1, rows, W), lambda b, r: (b, r, 0)),
        ],
        out_specs=pl.BlockSpec((1, 6, 128), lambda b, r: (b, 0, 0)),
        out_shape=jax.ShapeDtypeStruct((NB, 6, 128), jnp.float32),
    )(prediction, instances, labels)


def _pass2_body(bo, scal_ref, pred_ref, inst_ref, idx_ref, sfg_ref):
    b = pl.program_id(0) + bo
    k = pl.program_id(1)
    p = pred_ref[0]
    rows = p.shape[1]
    row0 = (k * rows).astype(jnp.float32)
    xm = lax.broadcasted_iota(jnp.int32, (rows, W), 1).astype(jnp.float32) * HX
    ym = (lax.broadcasted_iota(jnp.int32, (rows, W), 0).astype(jnp.float32) + row0) * HY
    ex = jnp.tanh(p[0]) + xm
    ey = jnp.tanh(p[1]) + ym
    sig = p[2]
    seed = jax.nn.sigmoid(p[3])
    inst = inst_ref[0]

    io = lax.broadcasted_iota(jnp.int32, (1, 128), 1)
    bf = jnp.float32(B)
    r2p = ex * ex + ey * ey
    down = jnp.zeros_like(ex)
    for i in range(NI):
        safe = jnp.maximum(scal_ref[b, 0, i], 1.0)
        cx = scal_ref[b, 1, i] / safe
        cy = scal_ref[b, 2, i] / safe
        s = jnp.exp(10.0 * scal_ref[b, 3, i] / safe)
        bx = -2.0 * s * cx
        by = -2.0 * s * cy
        c0 = s * (cx * cx + cy * cy)
        u = s * r2p + bx * ex + by * ey + c0
        d = jnp.exp(-u)
        own = inst == (i + 1)
        jn = jnp.minimum((bf * d).astype(jnp.int32), B - 1)
        idx_ref[0, i] = jnp.where(own, (i * 2 * B + B - 1) - jn,
                                  (i * 2 * B + B) + jn).astype(jnp.int16)
        down = down + jnp.where(own, d, 0.0)
    sfg = jnp.sum(jnp.where(inst > 0, (seed - down) ** 2, 0.0))
    sacc = jnp.where(io == 0, sfg, 0.0)

    @pl.when(k == 0)
    def _():
        sfg_ref[0] = sacc

    @pl.when(k != 0)
    def _():
        sfg_ref[0] = sfg_ref[0] + sacc


def _pass2(scal, prediction, instances, bo, nb):
    rows = 128
    nk = H // rows
    return pl.pallas_call(
        functools.partial(_pass2_body, bo),
        grid=(nb, nk),
        in_specs=[
            pl.BlockSpec(memory_space=pltpu.SMEM),
            pl.BlockSpec((1, 4, rows, W), lambda b, k: (b + bo, 0, k, 0)),
            pl.BlockSpec((1, rows, W), lambda b, k: (b + bo, k, 0)),
        ],
        out_specs=[
            pl.BlockSpec((1, NI, rows, W), lambda b, k: (b, 0, k, 0)),
            pl.BlockSpec((1, 1, 128), lambda b, k: (b, 0, 0)),
        ],
        out_shape=[
            jax.ShapeDtypeStruct((nb, NI, H, W), jnp.int16),
            jax.ShapeDtypeStruct((nb, 1, 128), jnp.float32),
        ],
    )(scal, prediction, instances)


def _pass3_body(*refs):
    nsplit = (len(refs) - 2) // 2
    parts_refs = refs[:nsplit]
    sums_ref = refs[nsplit]
    sfg_refs = refs[nsplit + 1:2 * nsplit + 1]
    out_ref = refs[2 * nsplit + 1]
    per = NB // nsplit
    slots = NW // per
    iar = lax.broadcasted_iota(jnp.int32, (B, B), 0)
    iac = lax.broadcasted_iota(jnp.int32, (B, B), 1)
    M = (iar >= iac).astype(jnp.float32)
    total = jnp.float32(0.0)
    for b in range(NB):
        parts_ref = parts_refs[b // per]
        sfg_ref = sfg_refs[b // per]
        bl = b % per
        tb1 = jnp.sum(parts_ref[bl * slots:(bl + 1) * slots], axis=0,
                      keepdims=True)
        tb = jnp.concatenate(
            [tb1[:, p * B:(p + 1) * B] for p in range(NPLANE)], axis=0)
        suf = jnp.dot(tb, M, preferred_element_type=jnp.float32)
        inst_loss = jnp.float32(0.0)
        var_loss = jnp.float32(0.0)
        obj = jnp.float32(0.0)
        seed_fg = sfg_ref[bl, 0]
        for i in range(NI):
            G = sums_ref[b, 0, i]
            pres = (G > 0.0).astype(jnp.float32)
            Gs = jnp.maximum(G, 1.0)
            C = suf[2 * i:2 * i + 1]
            Nn = suf[2 * i + 1:2 * i + 2]
            Nt = C + Nn
            J = 1.0 - (G - C) / jnp.maximum(G + Nt - C, 1.0)
            lov = (2.0 / B) * (jnp.sum(J) - 0.5 * J[0, 0])
            inst_loss = inst_loss + pres * lov
            ss = sums_ref[b, 3, i]
            ss2 = sums_ref[b, 4, i]
            mu = ss / Gs
            var_loss = var_loss + pres * (ss2 / Gs - mu * mu)
            obj = obj + pres
        denom = jnp.maximum(obj, 1.0)
        bg = sums_ref[b, 5, 0]
        seed_loss = (bg + seed_fg) / jnp.float32(NPIX)
        total = total + inst_loss / denom + 10.0 * var_loss / denom + seed_loss
    out_ref[0, 0] = total / NB


def _pass3(parts_list, sums, sfg_list):
    n = len(parts_list)
    return pl.pallas_call(
        _pass3_body,
        in_specs=(
            [pl.BlockSpec(memory_space=pltpu.VMEM)] * n
            + [pl.BlockSpec(memory_space=pltpu.SMEM)] * (n + 1)
        ),
        out_specs=pl.BlockSpec(memory_space=pltpu.SMEM),
        out_shape=jax.ShapeDtypeStruct((1, 1), jnp.float32),
    )(*parts_list, sums, *sfg_list)


NW = 32
CHROWS = 16
CHUNK = CHROWS * W


def _sc_hist_body(nbatch, idx_hbm, out_hbm, buf0, buf1, table, sem0, sem1):
    slots = NW // nbatch
    rows_per_slot = H // slots
    nch = NI * rows_per_slot // CHROWS
    cid = lax.axis_index("c")
    sid = lax.axis_index("s")
    wid = sid * 2 + cid
    batch = wid // slots
    slot = wid - batch * slots
    base_row = slot * rows_per_slot

    zeros = jnp.zeros((16,), jnp.float32)
    ones = jnp.ones((16,), jnp.float32)

    @plsc.parallel_loop(0, TBL // 16, unroll=8)
    def _(j):
        table[pl.ds(j * 16, 16)] = zeros

    def start(c, buf, sem):
        i = c // (rows_per_slot // CHROWS)
        rb = c - i * (rows_per_slot // CHROWS)
        pltpu.async_copy(
            idx_hbm.at[batch, i, pl.ds(base_row + rb * CHROWS, CHROWS)],
            buf, sem)

    def wait(buf, sem):
        pltpu.make_async_copy(
            idx_hbm.at[batch, 0, pl.ds(0, CHROWS)], buf, sem).wait()

    def process(buf):
        @plsc.parallel_loop(0, CHUNK // 32, unroll=8)
        def _(j):
            v = buf[j >> 4, pl.ds((j & 15) * 32, 32)]
            a, bb = plsc.unpack(v, format=plsc.PackFormat.INTERLEAVED)
            plsc.addupdate_scatter(table, [a], ones)
            plsc.addupdate_scatter(table, [bb], ones)

    start(0, buf0, sem0)

    def pair_body(p, carry):
        c0 = p * 2
        start(c0 + 1, buf1, sem1)
        wait(buf0, sem0)
        process(buf0)

        @pl.when(c0 + 2 < nch)
        def _():
            start(c0 + 2, buf0, sem0)
        wait(buf1, sem1)
        process(buf1)
        return carry
    lax.fori_loop(0, nch // 2, pair_body, 0)
    pltpu.sync_copy(table, out_hbm.at[wid])


def _sc_hist(idx_half, nbatch):
    mesh = plsc.VectorSubcoreMesh(core_axis_name="c", subcore_axis_name="s")
    f = functools.partial(
        pl.kernel,
        mesh=mesh,
        compiler_params=pltpu.CompilerParams(needs_layout_passes=False),
        out_type=jax.ShapeDtypeStruct((NW, TBL), jnp.float32),
        scratch_types=[
            pltpu.VMEM((CHROWS, W), jnp.int16),
            pltpu.VMEM((CHROWS, W), jnp.int16),
            pltpu.VMEM((TBL,), jnp.float32),
            pltpu.SemaphoreType.DMA,
            pltpu.SemaphoreType.DMA,
        ],
    )(functools.partial(_sc_hist_body, nbatch))
    return f(idx_half)


def kernel(prediction, instances, labels):
    sums = _pass1(prediction, instances, labels)
    nsplit = 2
    per = NB // nsplit
    parts_list, sfg_list = [], []
    for g in range(nsplit):
        idxg, sfgg = _pass2(sums, prediction, instances, g * per, per)
        parts_list.append(_sc_hist(idxg, per))
        sfg_list.append(sfgg[:, 0, :1])

    out = _pass3(parts_list, sums, sfg_list)
    return out.reshape(())
